# baseline (speedup 1.0000x reference)
import functools

import jax
import jax.numpy as jnp
from jax import lax
from jax.experimental import pallas as pl
from jax.experimental.pallas import tpu as pltpu
from jax.experimental.pallas import tpu_sc as plsc

B, T = 1024, 20
S_OBS, S_ACT = 8, 2
S_TOT = S_OBS + S_ACT
D = 64
NW = 32
PER = B // NW
N_OBS = T * S_OBS
N_ACT = T * S_ACT
RPB = T * S_TOT // 2
NBT = 8
LANES = 16


def _sc_body(obs_tok, act_tok, obs_tab, act_tab, out, refs):
    (ibo, iba, go, ga, sbuf, isem, gsem, osem) = refs
    wid = lax.axis_index("s") * 2 + lax.axis_index("c")
    base_b = wid * PER

    def fire_idx(i, p):
        b = base_b + i
        pltpu.async_copy(obs_tok.at[pl.ds(b * N_OBS, N_OBS)], ibo.at[p],
                         isem.at[p])
        pltpu.async_copy(act_tok.at[pl.ds(b * N_ACT, N_ACT)], iba.at[p],
                         isem.at[p])

    def wait_idx(i, p):
        b = base_b + i
        pltpu.make_async_copy(obs_tok.at[pl.ds(b * N_OBS, N_OBS)], ibo.at[p],
                              isem.at[p]).wait()
        pltpu.make_async_copy(act_tok.at[pl.ds(b * N_ACT, N_ACT)], iba.at[p],
                              isem.at[p]).wait()

    def fire_gather(p):
        pltpu.async_copy(obs_tab.at[ibo.at[p, pl.ds(0, 80)]],
                         go.at[p, pl.ds(0, 80), :], gsem.at[p])
        pltpu.async_copy(obs_tab.at[ibo.at[p, pl.ds(80, 80)]],
                         go.at[p, pl.ds(80, 80), :], gsem.at[p])
        pltpu.async_copy(act_tab.at[iba.at[p]], ga.at[p], gsem.at[p])

    def wait_gather(p):
        pltpu.make_async_copy(obs_tab.at[ibo.at[p, pl.ds(0, 80)]],
                              go.at[p, pl.ds(0, 80), :], gsem.at[p]).wait()
        pltpu.make_async_copy(obs_tab.at[ibo.at[p, pl.ds(80, 80)]],
                              go.at[p, pl.ds(80, 80), :], gsem.at[p]).wait()
        pltpu.make_async_copy(act_tab.at[iba.at[p]], ga.at[p],
                              gsem.at[p]).wait()

    def fire_out(i, r):
        b = base_b + i
        pltpu.async_copy(sbuf.at[r], out.at[pl.ds(b * RPB, RPB)], osem.at[r])

    def wait_out(i, r):
        b = base_b + i
        pltpu.make_async_copy(sbuf.at[r], out.at[pl.ds(b * RPB, RPB)],
                              osem.at[r]).wait()

    def repack(p, r):
        def do_t(t, c):
            for s in range(S_OBS):
                dr = t * (S_TOT // 2) + s // 2
                dc = (s % 2) * D
                for j in range(D // LANES):
                    sbuf[r, dr, pl.ds(dc + j * LANES, LANES)] = (
                        go[p, t * S_OBS + s, pl.ds(j * LANES, LANES)])
            for s in range(S_ACT):
                dr = t * (S_TOT // 2) + S_TOT // 2 - 1
                dc = s * D
                for j in range(D // LANES):
                    sbuf[r, dr, pl.ds(dc + j * LANES, LANES)] = (
                        ga[p, t * S_ACT + s, pl.ds(j * LANES, LANES)])
            return c

        lax.fori_loop(0, T, do_t, 0, unroll=2)

    fire_idx(0, 0)
    wait_idx(0, 0)
    fire_gather(0)
    fire_idx(1, 1)
    for i in (0, 1):
        p, r = i % 2, i % 3
        wait_gather(p)
        wait_idx(i + 1, 1 - p)
        fire_gather(1 - p)
        fire_idx(i + 2, p)
        repack(p, r)
        fire_out(i, r)

    def stage(i, c):
        p = lax.rem(i, 2)
        r = lax.rem(i, 3)
        wait_gather(p)
        wait_idx(i + 1, 1 - p)
        fire_gather(1 - p)
        fire_idx(i + 2, p)
        wait_out(i - 2, lax.rem(i - 2, 3))
        repack(p, r)
        fire_out(i, r)
        return c

    lax.fori_loop(2, PER - 2, stage, 0)

    i = PER - 2
    p, r = i % 2, i % 3
    wait_gather(p)
    wait_idx(i + 1, 1 - p)
    fire_gather(1 - p)
    wait_out(i - 2, (i - 2) % 3)
    repack(p, r)
    fire_out(i, r)

    i = PER - 1
    p, r = i % 2, i % 3
    wait_gather(p)
    wait_out(i - 2, (i - 2) % 3)
    repack(p, r)
    fire_out(i, r)

    wait_out(PER - 2, (PER - 2) % 3)
    wait_out(PER - 1, (PER - 1) % 3)


@functools.partial(
    pl.kernel,
    out_type=jax.ShapeDtypeStruct((B * RPB, 2 * D), jnp.float32),
    mesh=plsc.VectorSubcoreMesh(core_axis_name="c", subcore_axis_name="s",
                                num_cores=2),
    scratch_types=[
        pltpu.VMEM((2, N_OBS), jnp.int32),
        pltpu.VMEM((2, N_ACT), jnp.int32),
        pltpu.VMEM((2, N_OBS, D), jnp.float32),
        pltpu.VMEM((2, N_ACT, D), jnp.float32),
        pltpu.VMEM((3, RPB, 2 * D), jnp.float32),
        pltpu.SemaphoreType.DMA((2,)),
        pltpu.SemaphoreType.DMA((2,)),
        pltpu.SemaphoreType.DMA((3,)),
    ],
    compiler_params=pltpu.CompilerParams(use_tc_tiling_on_sc=False),
)
def _sc_gather(obs_tok, act_tok, obs_tab, act_tab, out, *refs):
    _sc_body(obs_tok, act_tok, obs_tab, act_tab, out, refs)


@functools.partial(
    pl.pallas_call,
    out_shape=jax.ShapeDtypeStruct((B, T * S_TOT, D), jnp.float32),
    grid=(B // NBT,),
    in_specs=[
        pl.BlockSpec((NBT * RPB, 2 * D), lambda i: (i, 0)),
        pl.BlockSpec((RPB, 2 * D), lambda i: (0, 0)),
    ],
    out_specs=pl.BlockSpec((NBT, T * S_TOT, D), lambda i: (i, 0, 0)),
    compiler_params=pltpu.CompilerParams(
        dimension_semantics=("parallel",)),
)
def _tc_finish(x_ref, pos_ref, out_ref):
    x = x_ref[...].reshape(NBT, RPB, 2 * D) + pos_ref[...][None]
    lo = x[..., 0:D]
    hi = x[..., D:2 * D]
    y = jnp.concatenate([lo[:, :, None, :], hi[:, :, None, :]], axis=2)
    out_ref[...] = y.reshape(NBT, T * S_TOT, D)


def kernel(obs_tokens, act_tokens, obs_table, act_table, pos_obs, pos_act,
           pos_ts):
    obs_tok = obs_tokens.reshape(B * T * S_OBS).astype(jnp.int32)
    act_tok = act_tokens.reshape(B * T * S_ACT).astype(jnp.int32)
    packed = _sc_gather(obs_tok, act_tok, obs_table, act_table)
    pos_full = jnp.concatenate(
        [pos_obs, pos_act], axis=0)[None, :, :] + pos_ts[:, None, :]
    pos128 = pos_full.reshape(RPB, 2 * D)
    return _tc_finish(packed, pos128)


# speedup vs baseline: 1.0874x; 1.0874x over previous
import functools

import jax
import jax.numpy as jnp
import numpy as np
from jax import lax
from jax.experimental import pallas as pl
from jax.experimental.pallas import tpu as pltpu
from jax.experimental.pallas import tpu_sc as plsc

B, T = 1024, 20
S_OBS, S_ACT = 8, 2
S_TOT = S_OBS + S_ACT
D = 64
NW = 32
PER = B // NW
NB = 2
NSTAGE = PER // NB
N_OBS = T * S_OBS
N_ACT = T * S_ACT
RO = NB * N_OBS
RA = NB * N_ACT
CH = 80
KO = RO // CH
KA = RA // CH
NBT = 16
LANES = 16


def _sc_body(obs_tok, act_tok, obs_tab, act_tab, pat_o, pat_a, out, refs):
    (ibo, iba, rows_o, rows_a, dio, dia, pat_ov, pat_av,
     isem, gsem, osem) = refs
    wid = lax.axis_index("s") * 2 + lax.axis_index("c")
    base_b = wid * PER

    pltpu.sync_copy(pat_o, pat_ov)
    pltpu.sync_copy(pat_a, pat_av)

    def fire_idx(i, p):
        b0 = base_b + i * NB
        pltpu.async_copy(obs_tok.at[pl.ds(b0 * N_OBS, RO)], ibo.at[p],
                         isem.at[p])
        pltpu.async_copy(act_tok.at[pl.ds(b0 * N_ACT, RA)], iba.at[p],
                         isem.at[p])

    def wait_idx(i, p):
        b0 = base_b + i * NB
        pltpu.make_async_copy(obs_tok.at[pl.ds(b0 * N_OBS, RO)], ibo.at[p],
                              isem.at[p]).wait()
        pltpu.make_async_copy(act_tok.at[pl.ds(b0 * N_ACT, RA)], iba.at[p],
                              isem.at[p]).wait()

    def fire_gather(p, r):
        for j in range(KO):
            pltpu.async_copy(
                obs_tab.at[ibo.at[p, pl.ds(j * CH, CH)]],
                rows_o.at[r, pl.ds(j * CH, CH), :], gsem.at[r])
        for j in range(KA):
            pltpu.async_copy(
                act_tab.at[iba.at[p, pl.ds(j * CH, CH)]],
                rows_a.at[r, pl.ds(j * CH, CH), :], gsem.at[r])

    def wait_gather(p, r):
        for j in range(KO):
            pltpu.make_async_copy(
                obs_tab.at[ibo.at[p, pl.ds(j * CH, CH)]],
                rows_o.at[r, pl.ds(j * CH, CH), :], gsem.at[r]).wait()
        for j in range(KA):
            pltpu.make_async_copy(
                act_tab.at[iba.at[p, pl.ds(j * CH, CH)]],
                rows_a.at[r, pl.ds(j * CH, CH), :], gsem.at[r]).wait()

    def fire_scatter(r):
        for j in range(KO):
            pltpu.async_copy(rows_o.at[r, pl.ds(j * CH, CH), :],
                             out.at[dio.at[r, j]], osem.at[r])
        for j in range(KA):
            pltpu.async_copy(rows_a.at[r, pl.ds(j * CH, CH), :],
                             out.at[dia.at[r, j]], osem.at[r])

    def wait_scatter(r):
        for j in range(KO):
            pltpu.make_async_copy(rows_o.at[r, pl.ds(j * CH, CH), :],
                                  out.at[dio.at[r, j]], osem.at[r]).wait()
        for j in range(KA):
            pltpu.make_async_copy(rows_a.at[r, pl.ds(j * CH, CH), :],
                                  out.at[dia.at[r, j]], osem.at[r]).wait()

    def compute_dst(i, r):
        base = (base_b + i * NB) * (T * S_TOT)
        for j in range(KO):
            for k in range(CH // LANES):
                sl = pl.ds(k * LANES, LANES)
                dio[r, j, sl] = pat_ov[j, sl] + base
        for j in range(KA):
            for k in range(CH // LANES):
                sl = pl.ds(k * LANES, LANES)
                dia[r, j, sl] = pat_av[j, sl] + base

    fire_idx(0, 0)
    wait_idx(0, 0)
    fire_gather(0, 0)
    fire_idx(1, 1)
    for i in range(NSTAGE):
        p = i % 2
        r = i % 3
        wait_gather(p, r)
        if i + 1 < NSTAGE:
            q, rn = (i + 1) % 2, (i + 1) % 3
            wait_idx(i + 1, q)
            if i >= 2:
                wait_scatter(rn)
            fire_gather(q, rn)
        if i + 2 < NSTAGE:
            fire_idx(i + 2, p)
        compute_dst(i, r)
        fire_scatter(r)
    if NSTAGE >= 2:
        wait_scatter((NSTAGE - 2) % 3)
    wait_scatter((NSTAGE - 1) % 3)


@functools.partial(
    pl.kernel,
    out_type=jax.ShapeDtypeStruct((B * T * S_TOT, D), jnp.float32),
    mesh=plsc.VectorSubcoreMesh(core_axis_name="c", subcore_axis_name="s",
                                num_cores=2),
    scratch_types=[
        pltpu.VMEM((2, RO), jnp.int32),
        pltpu.VMEM((2, RA), jnp.int32),
        pltpu.VMEM((3, RO, D), jnp.float32),
        pltpu.VMEM((3, RA, D), jnp.float32),
        pltpu.VMEM((3, KO, CH), jnp.int32),
        pltpu.VMEM((3, KA, CH), jnp.int32),
        pltpu.VMEM((KO, CH), jnp.int32),
        pltpu.VMEM((KA, CH), jnp.int32),
        pltpu.SemaphoreType.DMA((2,)),
        pltpu.SemaphoreType.DMA((3,)),
        pltpu.SemaphoreType.DMA((3,)),
    ],
    compiler_params=pltpu.CompilerParams(use_tc_tiling_on_sc=False),
)
def _sc_gather(obs_tok, act_tok, obs_tab, act_tab, pat_o, pat_a, out, *refs):
    _sc_body(obs_tok, act_tok, obs_tab, act_tab, pat_o, pat_a, out, refs)


def _dst_patterns():
    ro = np.arange(RO)
    po = ((ro // N_OBS) * (T * S_TOT) + ((ro % N_OBS) // S_OBS) * S_TOT
          + (ro % N_OBS) % S_OBS)
    ra = np.arange(RA)
    pa = ((ra // N_ACT) * (T * S_TOT) + ((ra % N_ACT) // S_ACT) * S_TOT
          + S_OBS + (ra % N_ACT) % S_ACT)
    return (po.reshape(KO, CH).astype(np.int32),
            pa.reshape(KA, CH).astype(np.int32))


_PAT_O, _PAT_A = _dst_patterns()


@functools.partial(
    pl.pallas_call,
    out_shape=jax.ShapeDtypeStruct((B, T * S_TOT, D), jnp.float32),
    grid=(B // NBT,),
    in_specs=[
        pl.BlockSpec((NBT * T * S_TOT, D), lambda i: (i, 0)),
        pl.BlockSpec((T * S_TOT, D), lambda i: (0, 0)),
    ],
    out_specs=pl.BlockSpec((NBT, T * S_TOT, D), lambda i: (i, 0, 0)),
    compiler_params=pltpu.CompilerParams(
        dimension_semantics=("parallel",)),
)
def _tc_finish(x_ref, pos_ref, out_ref):
    x = x_ref[...].reshape(NBT, T * S_TOT, D)
    out_ref[...] = x + pos_ref[...][None]


def kernel(obs_tokens, act_tokens, obs_table, act_table, pos_obs, pos_act,
           pos_ts):
    obs_tok = obs_tokens.reshape(B * T * S_OBS).astype(jnp.int32)
    act_tok = act_tokens.reshape(B * T * S_ACT).astype(jnp.int32)
    packed = _sc_gather(obs_tok, act_tok, obs_table, act_table,
                        jnp.asarray(_PAT_O), jnp.asarray(_PAT_A))
    pos_full = (jnp.concatenate([pos_obs, pos_act], axis=0)[None, :, :]
                + pos_ts[:, None, :]).reshape(T * S_TOT, D)
    return _tc_finish(packed, pos_full)


# speedup vs baseline: 1.4073x; 1.2942x over previous
import functools

import jax
import jax.numpy as jnp
import numpy as np
from jax import lax
from jax.experimental import pallas as pl
from jax.experimental.pallas import tpu as pltpu
from jax.experimental.pallas import tpu_sc as plsc

B, T = 1024, 20
S_OBS, S_ACT = 8, 2
S_TOT = S_OBS + S_ACT
D = 64
NW = 32
PER = B // NW
NB = 2
NSTAGE = PER // NB
N_OBS = T * S_OBS
N_ACT = T * S_ACT
RO = NB * N_OBS
RA = NB * N_ACT
CH = 80
KO = RO // CH
KA = RA // CH
LANES = 16


def _body(obs_tok, act_tok, obs_tab, act_tab, pos_o, pos_a, pat_o, pat_a,
          out, refs):
    (ibo, iba, rows_o, rows_a, dio, dia, pos_ov, pos_av, pat_ov, pat_av,
     isem, gsem, osem) = refs
    wid = lax.axis_index("s") * 2 + lax.axis_index("c")
    base_b = wid * PER

    pltpu.sync_copy(pos_o, pos_ov)
    pltpu.sync_copy(pos_a, pos_av)
    pltpu.sync_copy(pat_o, pat_ov)
    pltpu.sync_copy(pat_a, pat_av)

    def fire_idx(i, p):
        b0 = base_b + i * NB
        pltpu.async_copy(obs_tok.at[pl.ds(b0 * N_OBS, RO)], ibo.at[p],
                         isem.at[p])
        pltpu.async_copy(act_tok.at[pl.ds(b0 * N_ACT, RA)], iba.at[p],
                         isem.at[p])

    def wait_idx(i, p):
        b0 = base_b + i * NB
        pltpu.make_async_copy(obs_tok.at[pl.ds(b0 * N_OBS, RO)], ibo.at[p],
                              isem.at[p]).wait()
        pltpu.make_async_copy(act_tok.at[pl.ds(b0 * N_ACT, RA)], iba.at[p],
                              isem.at[p]).wait()

    def fire_gather(p, r):
        for j in range(KO):
            pltpu.async_copy(
                obs_tab.at[ibo.at[p, pl.ds(j * CH, CH)]],
                rows_o.at[r, pl.ds(j * CH, CH), :], gsem.at[r])
        for j in range(KA):
            pltpu.async_copy(
                act_tab.at[iba.at[p, pl.ds(j * CH, CH)]],
                rows_a.at[r, pl.ds(j * CH, CH), :], gsem.at[r])

    def wait_gather(p, r):
        for j in range(KO):
            pltpu.make_async_copy(
                obs_tab.at[ibo.at[p, pl.ds(j * CH, CH)]],
                rows_o.at[r, pl.ds(j * CH, CH), :], gsem.at[r]).wait()
        for j in range(KA):
            pltpu.make_async_copy(
                act_tab.at[iba.at[p, pl.ds(j * CH, CH)]],
                rows_a.at[r, pl.ds(j * CH, CH), :], gsem.at[r]).wait()

    def fire_scatter(r):
        for j in range(KO):
            pltpu.async_copy(rows_o.at[r, pl.ds(j * CH, CH), :],
                             out.at[dio.at[r, j]], osem.at[r])
        for j in range(KA):
            pltpu.async_copy(rows_a.at[r, pl.ds(j * CH, CH), :],
                             out.at[dia.at[r, j]], osem.at[r])

    def wait_scatter(r):
        for j in range(KO):
            pltpu.make_async_copy(rows_o.at[r, pl.ds(j * CH, CH), :],
                                  out.at[dio.at[r, j]], osem.at[r]).wait()
        for j in range(KA):
            pltpu.make_async_copy(rows_a.at[r, pl.ds(j * CH, CH), :],
                                  out.at[dia.at[r, j]], osem.at[r]).wait()

    def compute_stage(i, r):
        base = (base_b + i * NB) * (T * S_TOT)
        for j in range(KO):
            for k in range(CH // LANES):
                sl = pl.ds(k * LANES, LANES)
                dio[r, j, sl] = pat_ov[j, sl] + base
        for j in range(KA):
            for k in range(CH // LANES):
                sl = pl.ds(k * LANES, LANES)
                dia[r, j, sl] = pat_av[j, sl] + base

        def add_obs(q, c):
            for j in range(D // LANES):
                sl = pl.ds(j * LANES, LANES)
                pv = pos_ov[q, sl]
                for k in range(NB):
                    rows_o[r, k * N_OBS + q, sl] = (
                        rows_o[r, k * N_OBS + q, sl] + pv)
            return c

        def add_act(q, c):
            for j in range(D // LANES):
                sl = pl.ds(j * LANES, LANES)
                pv = pos_av[q, sl]
                for k in range(NB):
                    rows_a[r, k * N_ACT + q, sl] = (
                        rows_a[r, k * N_ACT + q, sl] + pv)
            return c

        lax.fori_loop(0, N_OBS, add_obs, 0, unroll=2)
        lax.fori_loop(0, N_ACT, add_act, 0, unroll=2)

    fire_idx(0, 0)
    wait_idx(0, 0)
    fire_gather(0, 0)
    fire_idx(1, 1)
    for i in range(NSTAGE):
        p = i % 2
        r = i % 3
        wait_gather(p, r)
        if i + 1 < NSTAGE:
            q, rn = (i + 1) % 2, (i + 1) % 3
            wait_idx(i + 1, q)
            if i >= 2:
                wait_scatter(rn)
            fire_gather(q, rn)
        if i + 2 < NSTAGE:
            fire_idx(i + 2, p)
        compute_stage(i, r)
        fire_scatter(r)
    if NSTAGE >= 2:
        wait_scatter((NSTAGE - 2) % 3)
    wait_scatter((NSTAGE - 1) % 3)


@functools.partial(
    pl.kernel,
    out_type=jax.ShapeDtypeStruct((B * T * S_TOT, D), jnp.float32),
    mesh=plsc.VectorSubcoreMesh(core_axis_name="c", subcore_axis_name="s",
                                num_cores=2),
    scratch_types=[
        pltpu.VMEM((2, RO), jnp.int32),
        pltpu.VMEM((2, RA), jnp.int32),
        pltpu.VMEM((3, RO, D), jnp.float32),
        pltpu.VMEM((3, RA, D), jnp.float32),
        pltpu.VMEM((3, KO, CH), jnp.int32),
        pltpu.VMEM((3, KA, CH), jnp.int32),
        pltpu.VMEM((N_OBS, D), jnp.float32),
        pltpu.VMEM((N_ACT, D), jnp.float32),
        pltpu.VMEM((KO, CH), jnp.int32),
        pltpu.VMEM((KA, CH), jnp.int32),
        pltpu.SemaphoreType.DMA((2,)),
        pltpu.SemaphoreType.DMA((3,)),
        pltpu.SemaphoreType.DMA((3,)),
    ],
    compiler_params=pltpu.CompilerParams(use_tc_tiling_on_sc=False),
)
def _episode_builder(obs_tok, act_tok, obs_tab, act_tab, pos_o, pos_a,
                     pat_o, pat_a, out, *refs):
    _body(obs_tok, act_tok, obs_tab, act_tab, pos_o, pos_a, pat_o, pat_a,
          out, refs)


def _dst_patterns():
    ro = np.arange(RO)
    po = ((ro // N_OBS) * (T * S_TOT) + ((ro % N_OBS) // S_OBS) * S_TOT
          + (ro % N_OBS) % S_OBS)
    ra = np.arange(RA)
    pa = ((ra // N_ACT) * (T * S_TOT) + ((ra % N_ACT) // S_ACT) * S_TOT
          + S_OBS + (ra % N_ACT) % S_ACT)
    return (po.reshape(KO, CH).astype(np.int32),
            pa.reshape(KA, CH).astype(np.int32))


_PAT_O, _PAT_A = _dst_patterns()


def kernel(obs_tokens, act_tokens, obs_table, act_table, pos_obs, pos_act,
           pos_ts):
    obs_tok = obs_tokens.reshape(B * T * S_OBS).astype(jnp.int32)
    act_tok = act_tokens.reshape(B * T * S_ACT).astype(jnp.int32)
    pos_o = (pos_obs[None, :, :] + pos_ts[:, None, :]).reshape(N_OBS, D)
    pos_a = (pos_act[None, :, :] + pos_ts[:, None, :]).reshape(N_ACT, D)
    out = _episode_builder(obs_tok, act_tok, obs_table, act_table,
                           pos_o, pos_a, jnp.asarray(_PAT_O),
                           jnp.asarray(_PAT_A))
    return out.reshape(B, T * S_TOT, D)
